# Initial kernel scaffold; baseline (speedup 1.0000x reference)
import jax
import jax.numpy as jnp
from jax.experimental import pallas as pl


def kernel(x, edge_index, edge_type, goalVec, goalObjectsVec, W0, Wg0, Ws0, b0, W1, Wg1, Ws1, b1, W2, Wg2, Ws2, b2, We, be, Wa, ba, fc1_W, fc1_b, fc2_W, fc2_b, fc3_W, fc3_b, fc4_W, fc4_b, fc5_W, fc5_b, tool_vec):
    raise NotImplementedError("write your pallas kernel here")


# speedup vs baseline: 28.3553x; 28.3553x over previous
import functools

import jax
import jax.numpy as jnp
from jax import lax
from jax.experimental import pallas as pl
from jax.experimental.pallas import tpu as pltpu
from jax.experimental.pallas import tpu_sc as plsc

_NC = 2
_NS = 16
_NW = _NC * _NS
_CH = 128


def _first_body(x_ref, W_ref, WgT_ref, Ws_ref, b_ref, y_ref, z_ref):
    h = x_ref[...]
    z_ref[...] = jnp.dot(h, Ws_ref[...], preferred_element_type=jnp.float32) + b_ref[...]
    for r in range(3):
        gate = jax.nn.sigmoid(
            jnp.sum(h * WgT_ref[r], axis=1, keepdims=True))
        y_ref[r] = gate * jnp.dot(h, W_ref[r], preferred_element_type=jnp.float32)


def _mid_body(agg_ref, zp_ref, W_ref, WgT_ref, Ws_ref, b_ref, y_ref, z_ref, h_ref):
    h = jnp.tanh(agg_ref[0] + agg_ref[1] + zp_ref[...])
    h_ref[...] = h
    z_ref[...] = jnp.dot(h, Ws_ref[...], preferred_element_type=jnp.float32) + b_ref[...]
    for r in range(3):
        gate = jax.nn.sigmoid(
            jnp.sum(h * WgT_ref[r], axis=1, keepdims=True))
        y_ref[r] = gate * jnp.dot(h, W_ref[r], preferred_element_type=jnp.float32)


def _table_first(x, W, WgT, Ws, b, blk):
    N, H = x.shape
    grid = N // blk
    return pl.pallas_call(
        _first_body,
        grid=(grid,),
        in_specs=[
            pl.BlockSpec((blk, H), lambda i: (i, 0)),
            pl.BlockSpec((3, H, H), lambda i: (0, 0, 0)),
            pl.BlockSpec((3, 1, H), lambda i: (0, 0, 0)),
            pl.BlockSpec((H, H), lambda i: (0, 0)),
            pl.BlockSpec((1, H), lambda i: (0, 0)),
        ],
        out_specs=[
            pl.BlockSpec((3, blk, H), lambda i: (0, i, 0)),
            pl.BlockSpec((blk, H), lambda i: (i, 0)),
        ],
        out_shape=[
            jax.ShapeDtypeStruct((3, N, H), jnp.float32),
            jax.ShapeDtypeStruct((N, H), jnp.float32),
        ],
    )(x, W, WgT, Ws, b)


def _table_mid(agg, zp, W, WgT, Ws, b, blk):
    N, H = zp.shape
    grid = N // blk
    return pl.pallas_call(
        _mid_body,
        grid=(grid,),
        in_specs=[
            pl.BlockSpec((2, blk, H), lambda i: (0, i, 0)),
            pl.BlockSpec((blk, H), lambda i: (i, 0)),
            pl.BlockSpec((3, H, H), lambda i: (0, 0, 0)),
            pl.BlockSpec((3, 1, H), lambda i: (0, 0, 0)),
            pl.BlockSpec((H, H), lambda i: (0, 0)),
            pl.BlockSpec((1, H), lambda i: (0, 0)),
        ],
        out_specs=[
            pl.BlockSpec((3, blk, H), lambda i: (0, i, 0)),
            pl.BlockSpec((blk, H), lambda i: (i, 0)),
            pl.BlockSpec((blk, H), lambda i: (i, 0)),
        ],
        out_shape=[
            jax.ShapeDtypeStruct((3, N, H), jnp.float32),
            jax.ShapeDtypeStruct((N, H), jnp.float32),
            jax.ShapeDtypeStruct((N, H), jnp.float32),
        ],
    )(agg, zp, W, WgT, Ws, b)


def _tail_body(agg_ref, zp_ref, We_ref, be_ref, Wa_ref, ba_ref,
               f1_ref, f1b_ref, f2_ref, f2b_ref, f3_ref, f3b_ref,
               f4_ref, f4b_ref, f5_ref, f5b_ref,
               gv_ref, gov_ref, tool_ref, out_ref):
    H = 128
    h3 = jnp.tanh(agg_ref[0] + agg_ref[1] + zp_ref[...])
    We = We_ref[...]
    be = be_ref[...]
    wa = Wa_ref[...]
    go = jnp.tanh(jnp.dot(gov_ref[...], We,
                          preferred_element_type=jnp.float32) + be)
    logits = jnp.dot(h3, wa[:H], preferred_element_type=jnp.float32)
    logits = logits + jnp.dot(go, wa[H:],
                              preferred_element_type=jnp.float32) + ba_ref[...]
    m = jnp.max(logits)
    w = jnp.exp(logits - m)
    attn = w / jnp.sum(w)
    scene = jnp.sum(attn * h3, axis=0, keepdims=True)
    ge = jnp.tanh(jnp.dot(gv_ref[...], We,
                          preferred_element_type=jnp.float32) + be)
    te = jnp.tanh(jnp.dot(tool_ref[...], We,
                          preferred_element_type=jnp.float32) + be)
    f1 = f1_ref[...]
    hh = jnp.tanh(jnp.dot(scene, f1[:H], preferred_element_type=jnp.float32)
                  + jnp.dot(ge, f1[H:2 * H], preferred_element_type=jnp.float32)
                  + jnp.dot(te, f1[2 * H:], preferred_element_type=jnp.float32)
                  + f1b_ref[...])
    hh = jnp.tanh(jnp.dot(hh, f2_ref[...], preferred_element_type=jnp.float32) + f2b_ref[...])
    hh = jnp.tanh(jnp.dot(hh, f3_ref[...], preferred_element_type=jnp.float32) + f3b_ref[...])
    hh = jnp.tanh(jnp.dot(hh, f4_ref[...], preferred_element_type=jnp.float32) + f4b_ref[...])
    out_ref[...] = jax.nn.sigmoid(
        jnp.dot(hh, f5_ref[...], preferred_element_type=jnp.float32) + f5b_ref[...])


def _tail(agg, zp, We, be, Wa, ba, fcs, gv, gov, tool_vec):
    T = tool_vec.shape[0]
    (f1, f1b), (f2, f2b), (f3, f3b), (f4, f4b), (f5, f5b) = fcs
    return pl.pallas_call(
        _tail_body,
        out_shape=jax.ShapeDtypeStruct((T, 1), jnp.float32),
    )(agg, zp, We, be, Wa, ba,
      f1, f1b, f2, f2b, f3, f3b, f4, f4b, f5, f5b,
      gv, gov, tool_vec)


def _sc_aggregate(y2d, flat3, dst3, zeros, N, H, n_chunks):
    rows_per_tile = ((N + _NS - 1) // _NS + 7) // 8 * 8
    NP = rows_per_tile * _NS

    mesh = plsc.VectorSubcoreMesh(core_axis_name="c", subcore_axis_name="s",
                                  num_cores=_NC)

    @functools.partial(
        pl.kernel,
        out_type=jax.ShapeDtypeStruct((_NC * NP, H), jnp.float32),
        mesh=mesh,
        scratch_types=[
            pltpu.VMEM((n_chunks, _CH), jnp.int32),
            pltpu.VMEM((n_chunks, _CH), jnp.int32),
            pltpu.VMEM((_CH, H), jnp.float32),
            pltpu.VMEM_SHARED((NP, H), jnp.float32),
            pltpu.SemaphoreType.DMA,
        ],
    )
    def k(y_hbm, flat_hbm, dst_hbm, zeros_hbm, out_hbm,
          idx_v, dst_v, rows_v, acc, sem):
        cid = lax.axis_index("c")
        sid = lax.axis_index("s")
        wid = sid * _NC + cid
        pltpu.sync_copy(zeros_hbm, acc.at[pl.ds(sid * rows_per_tile, rows_per_tile)])
        pltpu.sync_copy(flat_hbm.at[wid], idx_v)
        pltpu.sync_copy(dst_hbm.at[wid], dst_v)
        plsc.subcore_barrier()

        def chunk(j, carry):
            pltpu.async_copy(y_hbm.at[idx_v.at[j]], rows_v, sem).wait()
            pltpu.sync_copy(rows_v, acc.at[dst_v.at[j]], add=True)
            return carry

        lax.fori_loop(0, n_chunks, chunk, 0)
        plsc.subcore_barrier()
        pltpu.sync_copy(acc.at[pl.ds(sid * rows_per_tile, rows_per_tile)],
                        out_hbm.at[pl.ds(cid * NP + sid * rows_per_tile,
                                         rows_per_tile)])

    out = k(y2d, flat3, dst3, zeros)
    return out.reshape(_NC, NP, H)


def kernel(x, edge_index, edge_type, goalVec, goalObjectsVec,
           W0, Wg0, Ws0, b0, W1, Wg1, Ws1, b1, W2, Wg2, Ws2, b2,
           We, be, Wa, ba, fc1_W, fc1_b, fc2_W, fc2_b, fc3_W, fc3_b,
           fc4_W, fc4_b, fc5_W, fc5_b, tool_vec):
    N, H = x.shape[0], x.shape[1]
    E = edge_index.shape[1]
    assert E % _NW == 0
    ew = E // _NW
    n_chunks = (ew + _CH - 1) // _CH
    ewp = n_chunks * _CH
    rows_per_tile = ((N + _NS - 1) // _NS + 7) // 8 * 8
    NP = rows_per_tile * _NS

    src = edge_index[0]
    dst = edge_index[1]
    flat = edge_type * N + src
    pad = ewp - ew
    flat3 = jnp.pad(flat.reshape(_NW, ew), ((0, 0), (0, pad))
                    ).reshape(_NW, n_chunks, _CH)
    dst3 = jnp.pad(dst.reshape(_NW, ew), ((0, 0), (0, pad)),
                   constant_values=N).reshape(_NW, n_chunks, _CH)
    zeros = jnp.zeros((rows_per_tile, H), jnp.float32)

    blk = 1000
    WgT0 = jnp.transpose(Wg0, (0, 2, 1))
    WgT1 = jnp.transpose(Wg1, (0, 2, 1))
    WgT2 = jnp.transpose(Wg2, (0, 2, 1))
    b0r, b1r, b2r = b0.reshape(1, H), b1.reshape(1, H), b2.reshape(1, H)

    y, z = _table_first(x, W0, WgT0, Ws0, b0r, blk)
    agg = _sc_aggregate(y.reshape(3 * N, H), flat3, dst3, zeros, N, H, n_chunks)
    y, z, _h = _table_mid(agg[:, :N], z, W1, WgT1, Ws1, b1r, blk)
    agg = _sc_aggregate(y.reshape(3 * N, H), flat3, dst3, zeros, N, H, n_chunks)
    y, z, _h = _table_mid(agg[:, :N], z, W2, WgT2, Ws2, b2r, blk)
    agg = _sc_aggregate(y.reshape(3 * N, H), flat3, dst3, zeros, N, H, n_chunks)

    fcs = [(fc1_W, fc1_b.reshape(1, H)), (fc2_W, fc2_b.reshape(1, H)),
           (fc3_W, fc3_b.reshape(1, H)), (fc4_W, fc4_b.reshape(1, H)),
           (fc5_W, fc5_b.reshape(1, 1))]
    return _tail(agg[:, :N], z, We, be.reshape(1, H), Wa, ba.reshape(1, 1),
                 fcs, goalVec.reshape(1, -1), goalObjectsVec.reshape(1, -1),
                 tool_vec)
